# Initial kernel scaffold; baseline (speedup 1.0000x reference)
import functools

import jax
import jax.numpy as jnp
from jax import lax
from jax.experimental import pallas as pl
from jax.experimental.pallas import tpu as pltpu
from jax.experimental.pallas import tpu_sc as plsc

B = 1048576
D = 7
ROWS = 95
ROWS_PAD = 96
NC, NS, L = 2, 16, 16
NW = NC * NS
B_PER_W = B // NW
CHUNK = 4096
N_CHUNKS = B_PER_W // CHUNK
VECS = CHUNK // L


def _sc_gather(table_cols, idx):
    mesh = plsc.VectorSubcoreMesh(
        core_axis_name="c", subcore_axis_name="s", num_cores=NC, num_subcores=NS
    )

    @functools.partial(
        pl.kernel,
        out_type=jax.ShapeDtypeStruct((B * D,), jnp.float32),
        mesh=mesh,
        scratch_types=[
            [pltpu.VMEM((ROWS_PAD,), jnp.float32) for _ in range(D)],
            pltpu.VMEM((CHUNK,), jnp.int32),
            pltpu.VMEM((CHUNK * D,), jnp.float32),
        ],
    )
    def k(tab_hbm, idx_hbm, out_hbm, cols_v, idx_v, out_v):
        wid = lax.axis_index("s") * NC + lax.axis_index("c")
        base = wid * B_PER_W

        for c in range(D):
            pltpu.sync_copy(tab_hbm.at[c], cols_v[c])

        lane7 = lax.iota(jnp.int32, (L,)) * D

        def chunk_body(s, carry):
            start = base + s * CHUNK
            pltpu.sync_copy(idx_hbm.at[pl.ds(start, CHUNK)], idx_v)

            def vec_body(i, c2):
                z = idx_v[pl.ds(i * L, L)]
                opos = lane7 + i * (L * D)
                for c in range(D):
                    vals = plsc.load_gather(cols_v[c], [z])
                    plsc.store_scatter(out_v, [opos + c], vals)
                return c2

            lax.fori_loop(0, VECS, vec_body, 0, unroll=2)
            pltpu.sync_copy(out_v, out_hbm.at[pl.ds(start * D, CHUNK * D)])
            return carry

        lax.fori_loop(0, N_CHUNKS, chunk_body, 0)

    return k(table_cols, idx)


def kernel(atomic_numbers, table):
    idx = atomic_numbers.astype(jnp.int32)
    cols = jnp.zeros((D, ROWS_PAD), jnp.float32).at[:, :ROWS].set(table.T)
    out = _sc_gather(cols, idx)
    return out.reshape(B, D)


# speedup vs baseline: 5.8807x; 5.8807x over previous
import functools

import jax
import jax.numpy as jnp
from jax import lax
from jax.experimental import pallas as pl
from jax.experimental.pallas import tpu as pltpu
from jax.experimental.pallas import tpu_sc as plsc

B = 1048576
D = 7
ROWS = 95
ROWS_PAD = 96
NC, NS, L = 2, 16, 16
NW = NC * NS
B_PER_W = B // NW
CHUNK = 4096
N_CHUNKS = B_PER_W // CHUNK
VECS = CHUNK // L


def _sc_gather(table_cols, idx):
    mesh = plsc.VectorSubcoreMesh(
        core_axis_name="c", subcore_axis_name="s", num_cores=NC, num_subcores=NS
    )

    @functools.partial(
        pl.kernel,
        out_type=jax.ShapeDtypeStruct((B * D,), jnp.float32),
        mesh=mesh,
        scratch_types=[
            [pltpu.VMEM((ROWS_PAD,), jnp.float32) for _ in range(D)],
            pltpu.VMEM((CHUNK,), jnp.int32),
            pltpu.VMEM((CHUNK * D,), jnp.float32),
        ],
        compiler_params=pltpu.CompilerParams(needs_layout_passes=False),
    )
    def k(tab_hbm, idx_hbm, out_hbm, cols_v, idx_v, out_v):
        wid = lax.axis_index("s") * NC + lax.axis_index("c")
        base = wid * B_PER_W

        for c in range(D):
            pltpu.sync_copy(tab_hbm.at[pl.ds(c * ROWS_PAD, ROWS_PAD)], cols_v[c])

        lane7 = lax.iota(jnp.int32, L) * D

        def chunk_body(s, carry):
            start = base + s * CHUNK
            pltpu.sync_copy(idx_hbm.at[pl.ds(start, CHUNK)], idx_v)

            def vec_body(i, c2):
                z = idx_v[pl.ds(i * L, L)]
                opos = lane7 + i * (L * D)
                for c in range(D):
                    vals = plsc.load_gather(cols_v[c], [z])
                    plsc.store_scatter(out_v, [opos + c], vals)
                return c2

            lax.fori_loop(0, VECS, vec_body, 0, unroll=2)
            pltpu.sync_copy(out_v, out_hbm.at[pl.ds(start * D, CHUNK * D)])
            return carry

        lax.fori_loop(0, N_CHUNKS, chunk_body, 0)

    return k(table_cols, idx)


def kernel(atomic_numbers, table):
    idx = atomic_numbers.astype(jnp.int32)
    cols = (
        jnp.zeros((D, ROWS_PAD), jnp.float32).at[:, :ROWS].set(table.T).reshape(-1)
    )
    out = _sc_gather(cols, idx)
    return out.reshape(B, D)


# speedup vs baseline: 7.8418x; 1.3335x over previous
import functools

import jax
import jax.numpy as jnp
from jax import lax
from jax.experimental import pallas as pl
from jax.experimental.pallas import tpu as pltpu
from jax.experimental.pallas import tpu_sc as plsc

B = 1048576
D = 7
ROWS = 95
ROWS_PAD = 96
NC, NS, L = 2, 16, 16
NW = NC * NS
B_PER_W = B // NW
CHUNK = 4096
N_CHUNKS = B_PER_W // CHUNK
VECS = CHUNK // L


def _sc_gather(table_cols, idx):
    mesh = plsc.VectorSubcoreMesh(
        core_axis_name="c", subcore_axis_name="s", num_cores=NC, num_subcores=NS
    )

    @functools.partial(
        pl.kernel,
        out_type=jax.ShapeDtypeStruct((B, D), jnp.float32),
        mesh=mesh,
        scratch_types=[
            [pltpu.VMEM((ROWS_PAD,), jnp.float32) for _ in range(D)],
            pltpu.VMEM((CHUNK,), jnp.int32),
            pltpu.VMEM((CHUNK, D), jnp.float32),
        ],
        compiler_params=pltpu.CompilerParams(
            needs_layout_passes=False, use_tc_tiling_on_sc=False
        ),
    )
    def k(tab_hbm, idx_hbm, out_hbm, cols_v, idx_v, out_v):
        wid = lax.axis_index("s") * NC + lax.axis_index("c")
        base = wid * B_PER_W

        for c in range(D):
            pltpu.sync_copy(tab_hbm.at[pl.ds(c * ROWS_PAD, ROWS_PAD)], cols_v[c])

        lane = lax.iota(jnp.int32, L)
        cvecs = [jnp.full((L,), c, jnp.int32) for c in range(D)]

        def chunk_body(s, carry):
            start = base + s * CHUNK
            pltpu.sync_copy(idx_hbm.at[pl.ds(start, CHUNK)], idx_v)

            def vec_body(i, c2):
                z = idx_v[pl.ds(i * L, L)]
                row = lane + i * L
                for c in range(D):
                    vals = plsc.load_gather(cols_v[c], [z])
                    plsc.store_scatter(out_v, [row, cvecs[c]], vals)
                return c2

            lax.fori_loop(0, VECS, vec_body, 0, unroll=2)
            pltpu.sync_copy(out_v, out_hbm.at[pl.ds(start, CHUNK), :])
            return carry

        lax.fori_loop(0, N_CHUNKS, chunk_body, 0)

    return k(table_cols, idx)


def kernel(atomic_numbers, table):
    idx = atomic_numbers.astype(jnp.int32)
    cols = (
        jnp.zeros((D, ROWS_PAD), jnp.float32).at[:, :ROWS].set(table.T).reshape(-1)
    )
    return _sc_gather(cols, idx)
